# baseline (speedup 1.0000x reference)
import jax
import jax.numpy as jnp
from jax.experimental import pallas as pl
from jax.experimental.pallas import tpu as pltpu

_N = 1024
_NB = 128
_K = _N // _NB


def _local_layer_body(x_ref, adj_ref, w_ref, b_ref, o_ref,
                      a_sc, h_sc, deg_sc):
    k = pl.program_id(0)
    blk = adj_ref[...]
    a_sc[pl.ds(k * _NB, _NB), :] = blk.astype(jnp.bfloat16)
    psum = jnp.sum(blk, axis=0, keepdims=True)

    @pl.when(k == 0)
    def _init():
        deg_sc[...] = psum
        h_sc[...] = jnp.dot(x_ref[...], w_ref[...],
                            preferred_element_type=jnp.float32)

    @pl.when(k > 0)
    def _acc():
        deg_sc[...] += psum

    @pl.when(k == _K - 1)
    def _tail():
        dinv_r = jax.lax.rsqrt(deg_sc[...].astype(jnp.float32) + 1.0)
        dinv = jnp.transpose(dinv_r)
        scaled = h_sc[...] * dinv
        agg = jax.lax.dot_general(a_sc[...], scaled.astype(jnp.bfloat16),
                                  (((0,), (0,)), ((), ())),
                                  preferred_element_type=jnp.float32)
        out = (agg + scaled) * dinv + b_ref[...]
        o_ref[...] = jnp.where(out >= 0.0, out, 0.01 * out)


def kernel(x, adj, W, b):
    B, C, F_in = x.shape
    F_out = W.shape[1]
    x2d = x.reshape(_N, F_in)
    b2d = b.reshape(1, F_out)
    out = pl.pallas_call(
        _local_layer_body,
        grid=(_K,),
        in_specs=[
            pl.BlockSpec((_N, F_in), lambda k: (0, 0)),
            pl.BlockSpec((_NB, _N), lambda k: (k, 0)),
            pl.BlockSpec((F_in, F_out), lambda k: (0, 0)),
            pl.BlockSpec((1, F_out), lambda k: (0, 0)),
        ],
        out_specs=pl.BlockSpec((_N, F_out), lambda k: (0, 0)),
        scratch_shapes=[
            pltpu.VMEM((_N, _N), jnp.bfloat16),
            pltpu.VMEM((_N, F_out), jnp.float32),
            pltpu.VMEM((1, _N), jnp.int32),
        ],
        out_shape=jax.ShapeDtypeStruct((_N, F_out), x.dtype),
    )(x2d, adj, W, b2d)
    return out.reshape(B, C, F_out)


# speedup vs baseline: 1.3189x; 1.3189x over previous
import jax
import jax.numpy as jnp
from jax.experimental import pallas as pl
from jax.experimental.pallas import tpu as pltpu

_N = 1024
_NB = 256
_K = _N // _NB


def _local_layer_body(x_ref, adj_ref, w_ref, b_ref, o_ref,
                      a_sc, h_sc, deg_sc):
    k = pl.program_id(0)
    blk = adj_ref[...]
    a_sc[pl.ds(k * _NB, _NB), :] = blk.astype(jnp.bfloat16)
    psum = jnp.sum(blk, axis=0, keepdims=True)

    @pl.when(k == 0)
    def _init():
        deg_sc[...] = psum
        h_sc[...] = jnp.dot(x_ref[...], w_ref[...],
                            preferred_element_type=jnp.float32)

    @pl.when(k > 0)
    def _acc():
        deg_sc[...] += psum

    @pl.when(k == _K - 1)
    def _tail():
        dinv_r = jax.lax.rsqrt(deg_sc[...].astype(jnp.float32) + 1.0)
        dinv = jnp.transpose(dinv_r)
        scaled = h_sc[...] * dinv
        agg = jax.lax.dot_general(a_sc[...], scaled.astype(jnp.bfloat16),
                                  (((0,), (0,)), ((), ())),
                                  preferred_element_type=jnp.float32)
        out = (agg + scaled) * dinv + b_ref[...]
        o_ref[...] = jnp.where(out >= 0.0, out, 0.01 * out)


def kernel(x, adj, W, b):
    B, C, F_in = x.shape
    F_out = W.shape[1]
    x2d = x.reshape(_N, F_in)
    b2d = b.reshape(1, F_out)
    out = pl.pallas_call(
        _local_layer_body,
        grid=(_K,),
        in_specs=[
            pl.BlockSpec((_N, F_in), lambda k: (0, 0)),
            pl.BlockSpec((_NB, _N), lambda k: (k, 0)),
            pl.BlockSpec((F_in, F_out), lambda k: (0, 0)),
            pl.BlockSpec((1, F_out), lambda k: (0, 0)),
        ],
        out_specs=pl.BlockSpec((_N, F_out), lambda k: (0, 0)),
        scratch_shapes=[
            pltpu.VMEM((_N, _N), jnp.bfloat16),
            pltpu.VMEM((_N, F_out), jnp.float32),
            pltpu.VMEM((1, _N), jnp.int32),
        ],
        out_shape=jax.ShapeDtypeStruct((_N, F_out), x.dtype),
    )(x2d, adj, W, b2d)
    return out.reshape(B, C, F_out)
